# baseline (speedup 1.0000x reference)
import functools

import jax
import jax.numpy as jnp
from jax import lax
from jax.experimental import pallas as pl
from jax.experimental.pallas import tpu as pltpu
from jax.experimental.pallas import tpu_sc as plsc

_NEG = -1e30
_NC = 2
_NS = 16
_NW = _NC * _NS
_L = 16
_TOPK = 7


def _logits_kernel(x_ref, gw_ref, out_ref):
    out_ref[...] = lax.dot_general(
        gw_ref[...], x_ref[...], (((1,), (1,)), ((), ())),
        preferred_element_type=jnp.float32)


def _sc_route_body(lg_hbm, bias_hbm, zeros_hbm, rw_hbm, lg_v, bias_v, rw_v,
                   *, n_exp, tpw):
    wid = lax.axis_index("s") * _NC + lax.axis_index("c")
    pltpu.sync_copy(lg_hbm.at[wid], lg_v)
    pltpu.sync_copy(bias_hbm, bias_v)
    pltpu.sync_copy(zeros_hbm, rw_v)
    lanes = lax.iota(jnp.int32, _L)

    def group(g, carry):
        m = jnp.full((_L,), _NEG, jnp.float32)
        vs = []
        for r in range(1, n_exp):
            v = lg_v[g, r, :]
            vs.append(v)
            m = jnp.maximum(m, v)
        s = jnp.zeros((_L,), jnp.float32)
        ps = []
        for v in vs:
            p = jnp.exp(v - m)
            ps.append(p)
            s = s + p
        inv = 1.0 / s
        bs = [p * inv + bias_v[r + 1, :] for r, p in enumerate(ps)]
        idxs, vals = [], []
        tot = jnp.zeros((_L,), jnp.float32)
        for _ in range(_TOPK):
            mx = jnp.full((_L,), _NEG, jnp.float32)
            am = jnp.zeros((_L,), jnp.int32)
            for r in range(1, n_exp):
                cand = bs[r - 1]
                ok = cand > mx
                for prev in idxs:
                    ok = jnp.logical_and(ok, prev != r)
                mx = jnp.where(ok, cand, mx)
                am = jnp.where(ok, jnp.full((_L,), r, jnp.int32), am)
            idxs.append(am)
            vals.append(mx)
            tot = tot + mx
        scale = (_TOPK / (_TOPK + 1.0)) / tot
        rowbase = (g * _L + lanes) * n_exp
        plsc.store_scatter(rw_v, [rowbase],
                           jnp.full((_L,), 1.0 / (_TOPK + 1.0), jnp.float32))
        for k in range(_TOPK):
            plsc.store_scatter(rw_v, [rowbase + idxs[k]], vals[k] * scale)
        return carry

    lax.fori_loop(0, tpw // _L, group, 0)
    pltpu.sync_copy(rw_v, rw_hbm.at[pl.ds(wid * tpw * n_exp, tpw * n_exp)])


def _expert_kernel(rw_ref, x_ref, wfc_ref, wproj_ref, out_ref):
    e = pl.program_id(1)
    x = x_ref[...]
    h = lax.dot_general(x, wfc_ref[0], (((1,), (1,)), ((), ())),
                        preferred_element_type=jnp.float32)
    h = jnp.square(jnp.maximum(h, 0.0))
    y = lax.dot_general(h, wproj_ref[0], (((1,), (1,)), ((), ())),
                        preferred_element_type=jnp.float32)
    col = lax.broadcasted_iota(jnp.int32, rw_ref.shape, 1)
    w = jnp.sum(rw_ref[...] * (col == e).astype(jnp.float32), axis=1,
                keepdims=True)
    contrib = y * w

    @pl.when(e == 0)
    def _():
        out_ref[...] = contrib

    @pl.when(e != 0)
    def _():
        out_ref[...] += contrib


def kernel(x, c_fc_w, c_proj_w, gate_w, expert_bias):
    b, t, d = x.shape
    n_exp, h_dim, _ = c_fc_w.shape
    n_tok = b * t
    x_flat = x.reshape(n_tok, d)
    tpw = n_tok // _NW

    gw_pad = jnp.concatenate(
        [jnp.zeros((1, d), dtype=gate_w.dtype), gate_w], axis=0)
    bias_pad = jnp.concatenate(
        [jnp.full((1,), _NEG, dtype=expert_bias.dtype), expert_bias])
    bias_b = jnp.broadcast_to(bias_pad[:, None], (n_exp, _L))

    lt = min(1024, n_tok)
    logits = pl.pallas_call(
        _logits_kernel,
        grid=(n_tok // lt,),
        in_specs=[
            pl.BlockSpec((lt, d), lambda i: (i, 0)),
            pl.BlockSpec((n_exp, d), lambda i: (0, 0)),
        ],
        out_specs=pl.BlockSpec((n_exp, lt), lambda i: (0, i)),
        out_shape=jax.ShapeDtypeStruct((n_exp, n_tok), jnp.float32),
    )(x_flat, gw_pad)

    lg_blk = logits.reshape(n_exp, _NW, tpw // _L, _L).transpose(1, 2, 0, 3)
    zeros_rw = jnp.zeros((tpw * n_exp,), jnp.float32)

    sc_route = functools.partial(
        pl.kernel,
        out_type=jax.ShapeDtypeStruct((n_tok * n_exp,), jnp.float32),
        mesh=plsc.VectorSubcoreMesh(core_axis_name="c", subcore_axis_name="s",
                                    num_cores=_NC, num_subcores=_NS),
        scratch_types=[
            pltpu.VMEM((tpw // _L, n_exp, _L), jnp.float32),
            pltpu.VMEM((n_exp, _L), jnp.float32),
            pltpu.VMEM((tpw * n_exp,), jnp.float32),
        ],
        compiler_params=pltpu.CompilerParams(needs_layout_passes=False),
    )(functools.partial(_sc_route_body, n_exp=n_exp, tpw=tpw))
    rw = sc_route(lg_blk, bias_b, zeros_rw).reshape(n_tok, n_exp)

    tt = min(8192, n_tok)
    out = pl.pallas_call(
        _expert_kernel,
        grid=(n_tok // tt, n_exp),
        in_specs=[
            pl.BlockSpec((tt, n_exp), lambda i, e: (i, 0)),
            pl.BlockSpec((tt, d), lambda i, e: (i, 0)),
            pl.BlockSpec((1, h_dim, d), lambda i, e: (e, 0, 0)),
            pl.BlockSpec((1, d, h_dim), lambda i, e: (e, 0, 0)),
        ],
        out_specs=pl.BlockSpec((tt, d), lambda i, e: (i, 0)),
        out_shape=jax.ShapeDtypeStruct((n_tok, d), jnp.float32),
        compiler_params=pltpu.CompilerParams(
            dimension_semantics=("parallel", "arbitrary")),
    )(rw, x_flat, c_fc_w, c_proj_w)

    return out.reshape(b, t, d), rw
